# baseline (speedup 1.0000x reference)
import functools

import jax
import jax.numpy as jnp
from jax import lax
from jax.experimental import pallas as pl
from jax.experimental.pallas import tpu as pltpu
from jax.experimental.pallas import tpu_sc as plsc

VOCAB = 1000000
EMBD = 32
B = 4096
L = 200

NC = 2
NS = 16
NW = NC * NS

BT = B // 128
KU = 4
NGRP = L // KU


def _make_kernel():
    mesh = plsc.VectorSubcoreMesh(core_axis_name="c", subcore_axis_name="s")

    @functools.partial(
        pl.kernel,
        out_type=jax.ShapeDtypeStruct((L, EMBD // 8, BT, 8, 128), jnp.float32),
        mesh=mesh,
        compiler_params=pltpu.CompilerParams(
            use_tc_tiling_on_sc=False, needs_layout_passes=False
        ),
        scratch_types=[
            pltpu.VMEM((L, 128), jnp.int32),
            pltpu.VMEM((KU * 128, EMBD), jnp.float32),
            pltpu.VMEM((KU * 128, EMBD), jnp.float32),
            pltpu.VMEM((KU, EMBD // 8, 8, 128), jnp.float32),
            pltpu.VMEM((KU, EMBD // 8, 8, 128), jnp.float32),
            pltpu.SemaphoreType.DMA,
            pltpu.SemaphoreType.DMA,
            pltpu.SemaphoreType.DMA,
            pltpu.SemaphoreType.DMA,
        ],
    )
    def emb_kernel(table_hbm, xt_hbm, out_hbm,
                   idx_v, gb0, gb1, tb0, tb1, g0, g1, s0, s1):
        gbuf = (gb0, gb1)
        tbuf = (tb0, tb1)
        gsem = (g0, g1)
        ssem = (s0, s1)
        wid = lax.axis_index("s") * NC + lax.axis_index("c")
        pltpu.sync_copy(xt_hbm.at[:, pl.ds(wid * 128, 128)], idx_v)

        def gather_start(g, s):
            for u in range(KU):
                pltpu.async_copy(
                    table_hbm.at[idx_v.at[g * KU + u]],
                    gbuf[s].at[pl.ds(u * 128, 128)],
                    gsem[s],
                )

        def gather_wait(g, s):
            for u in range(KU):
                pltpu.make_async_copy(
                    table_hbm.at[idx_v.at[g * KU + u]],
                    gbuf[s].at[pl.ds(u * 128, 128)],
                    gsem[s],
                ).wait()

        def store_start(g, s):
            pltpu.async_copy(
                tbuf[s], out_hbm.at[pl.ds(g * KU, KU), :, wid], ssem[s]
            )

        def store_wait(g, s):
            pltpu.make_async_copy(
                tbuf[s], out_hbm.at[pl.ds(g * KU, KU), :, wid], ssem[s]
            ).wait()

        def transpose_relu(s):
            src = gbuf[s]
            dst = tbuf[s]
            lanes = lax.iota(jnp.int32, 16)
            for u in range(KU):

                @pl.loop(0, EMBD)
                def _feat(e):
                    cols = jnp.full((16,), e, jnp.int32)
                    for b16 in range(8):
                        rows = lanes + (u * 128 + b16 * 16)
                        vals = plsc.load_gather(src, [rows, cols])
                        dst[u, e // 8, e % 8, pl.ds(b16 * 16, 16)] = (
                            jnp.maximum(vals, 0.0)
                        )

        gather_start(0, 0)

        @pl.loop(0, NGRP, step=2)
        def _pair(G):
            for s in range(2):
                g = G + s
                o = 1 - s

                @pl.when(g >= 1)
                def _drain_prev_store():
                    store_wait(g - 1, o)

                @pl.when(g + 1 < NGRP)
                def _fire_next_gather():
                    gather_start(g + 1, o)

                gather_wait(g, s)
                transpose_relu(s)
                store_start(g, s)

        store_wait(NGRP - 1, 1)

    return emb_kernel


_EMB_KERNEL = _make_kernel()


@jax.jit
def kernel(x, table):
    out5 = _EMB_KERNEL(table, x.astype(jnp.int32).T)
    return out5.transpose(2, 4, 0, 1, 3).reshape(B, L, EMBD)
